# baseline (speedup 1.0000x reference)
import jax
import jax.numpy as jnp
from jax.experimental import pallas as pl
from jax.experimental.pallas import tpu as pltpu

_BN = 512
_BNR = 128


def _mhgcn_kernel(w_ref, feat_ref, w1_ref, b1_ref, w2_ref, b2_ref, a_ref,
                  out_ref, t_scr, s1_scr, s2_scr, u1_scr, x_scr):
    s = pl.program_id(0)
    ns = pl.num_programs(0)
    m, bnr, n = a_ref.shape
    nr = n // bnr
    bn = _BN

    @pl.when(s == 0)
    def _init():
        s1_scr[...] = jnp.dot(feat_ref[...], w1_ref[...],
                              preferred_element_type=jnp.float32
                              ).astype(jnp.bfloat16)
        u1_scr[...] = jnp.zeros_like(u1_scr)
        x_scr[...] = jnp.zeros_like(x_scr)

    @pl.when(s < nr)
    def _phase0():
        t_strip = a_ref[0] * w_ref[0]
        for k in range(1, m):
            t_strip += a_ref[k] * w_ref[k]
        tb = t_strip.astype(jnp.bfloat16)
        t_scr[pl.ds(s * bnr, bnr), :] = tb
        s1_s = s1_scr[pl.ds(s * bnr, bnr), :]
        u1_scr[pl.ds(s * bnr, bnr), :] += jnp.dot(
            tb, s1_scr[...], preferred_element_type=jnp.float32)
        u1_scr[...] += jax.lax.dot_general(
            tb, s1_s, (((0,), (0,)), ((), ())),
            preferred_element_type=jnp.float32)

    @pl.when(s == nr - 1)
    def _mid():
        u1_scr[...] += b1_ref[...]
        s2_scr[...] = jnp.dot(u1_scr[...], w2_ref[...],
                              preferred_element_type=jnp.float32
                              ).astype(jnp.bfloat16)

    @pl.when(s >= nr)
    def _phase1():
        k = s - nr
        strip = t_scr[pl.ds(k * bn, bn), :]
        s2_k = s2_scr[pl.ds(k * bn, bn), :]
        x_scr[pl.ds(k * bn, bn), :] += jnp.dot(
            strip, s2_scr[...], preferred_element_type=jnp.float32)
        x_scr[...] += jax.lax.dot_general(
            strip, s2_k, (((0,), (0,)), ((), ())),
            preferred_element_type=jnp.float32)

    @pl.when(s == ns - 1)
    def _fin():
        out_ref[...] = 0.5 * (u1_scr[...] + x_scr[...] + b2_ref[...])


@jax.jit
def kernel(feature, A, weight_b, W1, b1, W2, b2):
    n, f = feature.shape
    m = A.shape[0]
    o = W1.shape[1]
    bn = _BN
    nb = n // bn
    bnr = _BNR
    nr = n // bnr

    w = weight_b.reshape(m)
    b1r = b1.reshape(1, o)
    b2r = b2.reshape(1, o)

    def a_map(s):
        return (0, jnp.minimum(s, nr - 1), 0)

    out = pl.pallas_call(
        _mhgcn_kernel,
        grid=(nr + nb,),
        in_specs=[
            pl.BlockSpec(memory_space=pltpu.SMEM),
            pl.BlockSpec((n, f), lambda s: (0, 0)),
            pl.BlockSpec((f, o), lambda s: (0, 0)),
            pl.BlockSpec((1, o), lambda s: (0, 0)),
            pl.BlockSpec((o, o), lambda s: (0, 0)),
            pl.BlockSpec((1, o), lambda s: (0, 0)),
            pl.BlockSpec((m, bnr, n), a_map),
        ],
        out_specs=pl.BlockSpec((n, o), lambda s: (0, 0)),
        out_shape=jax.ShapeDtypeStruct((n, o), jnp.float32),
        compiler_params=pltpu.CompilerParams(
            fuse_transposed_lhs_in_matmul=True),
        scratch_shapes=[
            pltpu.VMEM((n, n), jnp.bfloat16),
            pltpu.VMEM((n, o), jnp.bfloat16),
            pltpu.VMEM((n, o), jnp.bfloat16),
            pltpu.VMEM((n, o), jnp.float32),
            pltpu.VMEM((n, o), jnp.float32),
        ],
    )(w, feature, W1, b1r, W2, b2r, A)

    return out


# speedup vs baseline: 1.0728x; 1.0728x over previous
import jax
import jax.numpy as jnp
from jax.experimental import pallas as pl
from jax.experimental.pallas import tpu as pltpu

_BN = 512
_BNR = 128


def _mhgcn_kernel(w_ref, feat_ref, w1_ref, b1_ref, w2_ref, b2_ref, a_ref,
                  out_ref, t_scr, s1_scr, s2_scr, u1_scr, x_scr):
    s = pl.program_id(0)
    ns = pl.num_programs(0)
    m, bnr, n = a_ref.shape
    nr = n // bnr
    bn = _BN

    @pl.when(s == 0)
    def _init():
        s1_scr[...] = jnp.dot(feat_ref[...], w1_ref[...],
                              preferred_element_type=jnp.float32
                              ).astype(jnp.bfloat16)
        u1_scr[...] = jnp.zeros_like(u1_scr)
        x_scr[...] = jnp.zeros_like(x_scr)

    @pl.when(s < nr)
    def _phase0():
        t_strip = a_ref[0] * w_ref[0]
        for k in range(1, m):
            t_strip += a_ref[k] * w_ref[k]
        tb = t_strip.astype(jnp.bfloat16)
        t_scr[pl.ds(s * bnr, bnr), :] = tb
        s1_s = s1_scr[pl.ds(s * bnr, bnr), :]
        u1_scr[pl.ds(s * bnr, bnr), :] += jnp.dot(
            tb, s1_scr[...], preferred_element_type=jnp.float32)
        u1_scr[...] += jax.lax.dot_general(
            tb, s1_s, (((0,), (0,)), ((), ())),
            preferred_element_type=jnp.float32)

    @pl.when(s == nr - 1)
    def _mid():
        u1_scr[...] += b1_ref[...]
        s2_scr[...] = jnp.dot(u1_scr[...], w2_ref[...],
                              preferred_element_type=jnp.float32
                              ).astype(jnp.bfloat16)

    @pl.when(s >= nr)
    def _phase1():
        k = s - nr
        strip = t_scr[pl.ds(k * bn, bn), :]
        s2_k = s2_scr[pl.ds(k * bn, bn), :]
        x_scr[pl.ds(k * bn, bn), :] += jnp.dot(
            strip, s2_scr[...], preferred_element_type=jnp.float32)
        x_scr[...] += jax.lax.dot_general(
            strip, s2_k, (((0,), (0,)), ((), ())),
            preferred_element_type=jnp.float32)

    @pl.when(s == ns - 1)
    def _fin():
        out_ref[...] = 0.5 * (u1_scr[...] + x_scr[...] + b2_ref[...])


@jax.jit
def kernel(feature, A, weight_b, W1, b1, W2, b2):
    n, f = feature.shape
    m = A.shape[0]
    o = W1.shape[1]
    bn = _BN
    nb = n // bn
    bnr = _BNR
    nr = n // bnr

    w = weight_b.reshape(m)
    b1r = b1.reshape(1, o)
    b2r = b2.reshape(1, o)

    def a_map(s):
        return (0, jnp.minimum(s, nr - 1), 0)

    out = pl.pallas_call(
        _mhgcn_kernel,
        grid=(nr + nb,),
        in_specs=[
            pl.BlockSpec(memory_space=pltpu.SMEM),
            pl.BlockSpec((n, f), lambda s: (0, 0)),
            pl.BlockSpec((f, o), lambda s: (0, 0)),
            pl.BlockSpec((1, o), lambda s: (0, 0)),
            pl.BlockSpec((o, o), lambda s: (0, 0)),
            pl.BlockSpec((1, o), lambda s: (0, 0)),
            pl.BlockSpec((m, bnr, n), a_map),
        ],
        out_specs=pl.BlockSpec((n, o), lambda s: (0, 0)),
        out_shape=jax.ShapeDtypeStruct((n, o), jnp.float32),
        scratch_shapes=[
            pltpu.VMEM((n, n), jnp.bfloat16),
            pltpu.VMEM((n, o), jnp.bfloat16),
            pltpu.VMEM((n, o), jnp.bfloat16),
            pltpu.VMEM((n, o), jnp.float32),
            pltpu.VMEM((n, o), jnp.float32),
        ],
    )(w, feature, W1, b1r, W2, b2r, A)

    return out
